# baseline (speedup 1.0000x reference)
import jax
import jax.numpy as jnp
from jax import lax
from jax.experimental import pallas as pl
from jax.experimental.pallas import tpu as pltpu
from jax.experimental.pallas import tpu_sc as plsc

_NC = 2
_NS = 16
_NW = _NC * _NS

_PIECE = 32768
_CHUNK = 2048
_LANES = 16
_SUB = 512


def kernel(fixed_values, refinable_params, refinable_idx):
    n = fixed_values.shape[0]
    r = refinable_params.shape[0]
    n_pieces = n // _PIECE
    pieces_per_w = n_pieces // _NW
    n_sub = r // _SUB

    sub = refinable_idx[::_SUB]

    mesh = plsc.VectorSubcoreMesh(
        core_axis_name="c", subcore_axis_name="s",
        num_cores=_NC, num_subcores=_NS,
    )

    @pl.kernel(
        mesh=mesh,
        out_type=jax.ShapeDtypeStruct((n,), jnp.float32),
        compiler_params=pltpu.CompilerParams(needs_layout_passes=False),
        scratch_types=[
            pltpu.VMEM((_PIECE,), jnp.float32),
            pltpu.VMEM((_PIECE,), jnp.float32),
            pltpu.VMEM((_PIECE,), jnp.float32),
            pltpu.VMEM((_CHUNK,), jnp.int32),
            pltpu.VMEM((_CHUNK,), jnp.float32),
            pltpu.VMEM((1, n_sub), jnp.int32),
            pltpu.VMEM((2 * pieces_per_w, _SUB), jnp.int32),
            pltpu.SemaphoreType.DMA,
            pltpu.SemaphoreType.DMA,
            pltpu.SemaphoreType.DMA,
            pltpu.SemaphoreType.DMA,
            pltpu.SemaphoreType.DMA,
            pltpu.SemaphoreType.DMA,
            pltpu.SemaphoreType.DMA,
        ],
    )
    def sc_merge(fixed_hbm, idx_hbm, prm_hbm, sub_hbm, out_hbm,
                 buf0, buf1, buf2, idx_v, prm_v, sub2_v, win_v, sem,
                 sem_in0, sem_in1, sem_in2, sem_out0, sem_out1, sem_out2):
        wid = lax.axis_index("s") * _NC + lax.axis_index("c")
        pltpu.sync_copy(sub_hbm, sub2_v)

        n_b = 2 * pieces_per_w
        bvals = []
        for k in range(pieces_per_w):
            p = wid + k * _NW
            bvals.append(p * _PIECE)
            bvals.append((p + 1) * _PIECE)

        zero = jnp.int32(0)
        n_sv = n_sub // _LANES

        def count_sorted(ref, row, n_vecs, q0, bval):
            def cond(q):
                qc = lax.min(q, n_vecs - 1)
                v = ref[row, pl.ds(qc * _LANES, _LANES)]
                return (q < n_vecs) & (v[_LANES - 1] < bval)

            q = lax.while_loop(cond, lambda q: q + 1, q0)
            qc = lax.min(q, n_vecs - 1)
            v = ref[row, pl.ds(qc * _LANES, _LANES)]
            cnt = qc * _LANES + plsc.all_reduce_population_count(v < bval)[0]
            return q, cnt

        c1 = []
        q = zero
        for j in range(n_b):
            q, cnt = count_sorted(sub2_v, 0, n_sv, q, bvals[j])
            c1.append(cnt)
        offs = [lax.max(c1[j] - 1, zero) * _SUB for j in range(n_b)]

        for j in range(n_b):
            pltpu.async_copy(
                idx_hbm.at[pl.ds(pl.multiple_of(offs[j], 8), _SUB)],
                win_v.at[j], sem)
        for j in range(n_b):
            pltpu.make_async_copy(
                idx_hbm.at[pl.ds(0, _SUB)], win_v.at[0], sem).wait()

        bounds = []
        for j in range(n_b):
            _, cnt = count_sorted(win_v, j, _SUB // _LANES, zero, bvals[j])
            bounds.append(offs[j] + cnt)

        sem_outs = [sem_out0, sem_out1, sem_out2]
        sem_ins = [sem_in0, sem_in1, sem_in2]
        ring = [buf0, buf1, buf2]
        _R = len(ring)

        def drain_out(slot):
            pltpu.make_async_copy(ring[slot], out_hbm.at[pl.ds(0, _PIECE)],
                                  sem_outs[slot]).wait()

        def piece_args(k):
            p = wid + k * _NW
            plo = p * _PIECE
            a = bounds[2 * k]
            b = bounds[2 * k + 1]
            cnt = b - a
            is_full = (cnt == _PIECE) & ((a & 7) == 0)
            return plo, a, b, cnt, is_full

        def fire_in(k):
            plo, a, b, cnt, is_full = piece_args(k)
            buf, s = ring[k % _R], sem_ins[k % _R]

            @pl.when(is_full)
            def _():
                pltpu.async_copy(
                    prm_hbm.at[pl.ds(pl.multiple_of(a, 8), _PIECE)], buf, s)

            @pl.when(jnp.logical_not(is_full))
            def _():
                pltpu.async_copy(fixed_hbm.at[pl.ds(plo, _PIECE)], buf, s)

        fire_in(0)
        for k in range(pieces_per_w):
            plo, a, b, cnt, is_full = piece_args(k)
            phi = plo + _PIECE
            buf = ring[k % _R]

            if k + 1 < pieces_per_w:
                if k + 1 >= _R:
                    drain_out((k + 1) % _R)
                fire_in(k + 1)

            pltpu.make_async_copy(fixed_hbm.at[pl.ds(0, _PIECE)], buf,
                                  sem_ins[k % _R]).wait()

            @pl.when((cnt > 0) & jnp.logical_not(is_full))
            def _():
                a_r = a & ~7
                n_chunks = (b - a_r + _CHUNK - 1) // _CHUNK

                def do_chunk(c, carry2):
                    base = pl.multiple_of(
                        lax.min(a_r + c * _CHUNK, r - _CHUNK), 8)
                    pltpu.sync_copy(idx_hbm.at[pl.ds(base, _CHUNK)], idx_v)
                    pltpu.sync_copy(prm_hbm.at[pl.ds(base, _CHUNK)], prm_v)
                    rem = b - base
                    n_vec = lax.min((rem + _LANES - 1) // _LANES,
                                    _CHUNK // _LANES)

                    def do_vec(v, carry3):
                        iv = idx_v[pl.ds(v * _LANES, _LANES)]
                        pv = prm_v[pl.ds(v * _LANES, _LANES)]
                        mask = (iv >= plo) & (iv < phi)
                        plsc.store_scatter(buf, [iv - plo], pv, mask=mask)
                        return carry3

                    lax.fori_loop(0, n_vec, do_vec, 0)
                    return carry2

                lax.fori_loop(0, n_chunks, do_chunk, 0)

            pltpu.async_copy(buf, out_hbm.at[pl.ds(plo, _PIECE)],
                             sem_outs[k % _R])

        for k in range(max(0, pieces_per_w - _R), pieces_per_w):
            drain_out(k % _R)

    return sc_merge(fixed_values, refinable_idx, refinable_params,
                    sub.reshape(1, n_sub))


# speedup vs baseline: 1.1038x; 1.1038x over previous
import jax
import jax.numpy as jnp
from jax import lax
from jax.experimental import pallas as pl
from jax.experimental.pallas import tpu as pltpu
from jax.experimental.pallas import tpu_sc as plsc

_NC = 2
_NS = 16
_NW = _NC * _NS

_PIECE = 32768
_CHUNK = 2048
_LANES = 16
_SUB = 512


def kernel(fixed_values, refinable_params, refinable_idx):
    n = fixed_values.shape[0]
    r = refinable_params.shape[0]
    n_pieces = n // _PIECE
    pieces_per_w = n_pieces // _NW
    n_sub = r // _SUB

    sub = refinable_idx[::_SUB]

    mesh = plsc.VectorSubcoreMesh(
        core_axis_name="c", subcore_axis_name="s",
        num_cores=_NC, num_subcores=_NS,
    )

    @pl.kernel(
        mesh=mesh,
        out_type=jax.ShapeDtypeStruct((n,), jnp.float32),
        compiler_params=pltpu.CompilerParams(needs_layout_passes=False),
        scratch_types=[
            pltpu.VMEM((_PIECE,), jnp.float32),
            pltpu.VMEM((_PIECE,), jnp.float32),
            pltpu.VMEM((_PIECE,), jnp.float32),
            pltpu.VMEM((_CHUNK,), jnp.int32),
            pltpu.VMEM((_CHUNK,), jnp.float32),
            pltpu.VMEM((n_sub,), jnp.int32),
            pltpu.VMEM((2 * pieces_per_w, _SUB), jnp.int32),
            pltpu.SemaphoreType.DMA,
            pltpu.SemaphoreType.DMA,
            pltpu.SemaphoreType.DMA,
            pltpu.SemaphoreType.DMA,
            pltpu.SemaphoreType.DMA,
            pltpu.SemaphoreType.DMA,
            pltpu.SemaphoreType.DMA,
        ],
    )
    def sc_merge(fixed_hbm, idx_hbm, prm_hbm, sub_hbm, out_hbm,
                 buf0, buf1, buf2, idx_v, prm_v, sub_v, win_v, sem,
                 sem_in0, sem_in1, sem_in2, sem_out0, sem_out1, sem_out2):
        wid = lax.axis_index("s") * _NC + lax.axis_index("c")
        pltpu.sync_copy(sub_hbm, sub_v)

        n_b = 2 * pieces_per_w
        bvals = []
        for k in range(pieces_per_w):
            p = wid + k * _NW
            bvals.append(p * _PIECE)
            bvals.append((p + 1) * _PIECE)

        zero = jnp.int32(0)

        def bin_count(load, n_vecs, bval):
            q = zero
            sz = n_vecs // 2
            while sz >= 1:
                v = load(q + (sz - 1))
                q = q + jnp.where(v[_LANES - 1] < bval, jnp.int32(sz), zero)
                sz //= 2
            v = load(q)
            return q * _LANES + plsc.all_reduce_population_count(v < bval)[0]

        c1 = [
            bin_count(lambda q: sub_v[pl.ds(q * _LANES, _LANES)],
                      n_sub // _LANES, bvals[j])
            for j in range(n_b)
        ]
        offs = [lax.max(c1[j] - 1, zero) * _SUB for j in range(n_b)]

        for j in range(n_b):
            pltpu.async_copy(
                idx_hbm.at[pl.ds(pl.multiple_of(offs[j], 8), _SUB)],
                win_v.at[j], sem)
        for j in range(n_b):
            pltpu.make_async_copy(
                idx_hbm.at[pl.ds(0, _SUB)], win_v.at[0], sem).wait()

        bounds = []
        for j in range(n_b):
            cnt = bin_count(
                lambda q, j=j: win_v[j, pl.ds(q * _LANES, _LANES)],
                _SUB // _LANES, bvals[j])
            bounds.append(offs[j] + cnt)

        sem_outs = [sem_out0, sem_out1, sem_out2]
        sem_ins = [sem_in0, sem_in1, sem_in2]
        ring = [buf0, buf1, buf2]
        _R = len(ring)

        def drain_out(slot):
            pltpu.make_async_copy(ring[slot], out_hbm.at[pl.ds(0, _PIECE)],
                                  sem_outs[slot]).wait()

        def piece_args(k):
            p = wid + k * _NW
            plo = p * _PIECE
            a = bounds[2 * k]
            b = bounds[2 * k + 1]
            cnt = b - a
            is_full = (cnt == _PIECE) & ((a & 7) == 0)
            return plo, a, b, cnt, is_full

        def fire_in(k):
            plo, a, b, cnt, is_full = piece_args(k)
            buf, s = ring[k % _R], sem_ins[k % _R]

            @pl.when(is_full)
            def _():
                pltpu.async_copy(
                    prm_hbm.at[pl.ds(pl.multiple_of(a, 8), _PIECE)], buf, s)

            @pl.when(jnp.logical_not(is_full))
            def _():
                pltpu.async_copy(fixed_hbm.at[pl.ds(plo, _PIECE)], buf, s)

        fire_in(0)
        for k in range(pieces_per_w):
            plo, a, b, cnt, is_full = piece_args(k)
            phi = plo + _PIECE
            buf = ring[k % _R]

            if k + 1 < pieces_per_w:
                if k + 1 >= _R:
                    drain_out((k + 1) % _R)
                fire_in(k + 1)

            pltpu.make_async_copy(fixed_hbm.at[pl.ds(0, _PIECE)], buf,
                                  sem_ins[k % _R]).wait()

            @pl.when((cnt > 0) & jnp.logical_not(is_full))
            def _():
                a_r = a & ~7
                n_chunks = (b - a_r + _CHUNK - 1) // _CHUNK

                def do_chunk(c, carry2):
                    base = pl.multiple_of(
                        lax.min(a_r + c * _CHUNK, r - _CHUNK), 8)
                    pltpu.sync_copy(idx_hbm.at[pl.ds(base, _CHUNK)], idx_v)
                    pltpu.sync_copy(prm_hbm.at[pl.ds(base, _CHUNK)], prm_v)
                    rem = b - base
                    n_vec = lax.min((rem + _LANES - 1) // _LANES,
                                    _CHUNK // _LANES)

                    def do_vec(v, carry3):
                        iv = idx_v[pl.ds(v * _LANES, _LANES)]
                        pv = prm_v[pl.ds(v * _LANES, _LANES)]
                        mask = (iv >= plo) & (iv < phi)
                        plsc.store_scatter(buf, [iv - plo], pv, mask=mask)
                        return carry3

                    lax.fori_loop(0, n_vec, do_vec, 0)
                    return carry2

                lax.fori_loop(0, n_chunks, do_chunk, 0)

            pltpu.async_copy(buf, out_hbm.at[pl.ds(plo, _PIECE)],
                             sem_outs[k % _R])

        for k in range(max(0, pieces_per_w - _R), pieces_per_w):
            drain_out(k % _R)

    return sc_merge(fixed_values, refinable_idx, refinable_params, sub)


# speedup vs baseline: 1.1320x; 1.0256x over previous
import jax
import jax.numpy as jnp
from jax import lax
from jax.experimental import pallas as pl
from jax.experimental.pallas import tpu as pltpu
from jax.experimental.pallas import tpu_sc as plsc

_NC = 2
_NS = 16
_NW = _NC * _NS

_PIECE = 32768
_CHUNK = 2048
_LANES = 16
_SUB = 512


def kernel(fixed_values, refinable_params, refinable_idx):
    n = fixed_values.shape[0]
    r = refinable_params.shape[0]
    n_pieces = n // _PIECE
    pieces_per_w = n_pieces // _NW
    n_sub = r // _SUB

    sub = refinable_idx[::_SUB]

    mesh = plsc.VectorSubcoreMesh(
        core_axis_name="c", subcore_axis_name="s",
        num_cores=_NC, num_subcores=_NS,
    )

    @pl.kernel(
        mesh=mesh,
        out_type=jax.ShapeDtypeStruct((n,), jnp.float32),
        compiler_params=pltpu.CompilerParams(needs_layout_passes=False),
        scratch_types=[
            pltpu.VMEM((_PIECE,), jnp.float32),
            pltpu.VMEM((_PIECE,), jnp.float32),
            pltpu.VMEM((_PIECE,), jnp.float32),
            pltpu.VMEM((_CHUNK,), jnp.int32),
            pltpu.VMEM((_CHUNK,), jnp.float32),
            pltpu.VMEM((n_sub,), jnp.int32),
            pltpu.VMEM((2 * pieces_per_w, _SUB), jnp.int32),
            pltpu.SemaphoreType.DMA,
            pltpu.SemaphoreType.DMA,
            pltpu.SemaphoreType.DMA,
            pltpu.SemaphoreType.DMA,
            pltpu.SemaphoreType.DMA,
            pltpu.SemaphoreType.DMA,
            pltpu.SemaphoreType.DMA,
        ],
    )
    def sc_merge(fixed_hbm, idx_hbm, prm_hbm, sub_hbm, out_hbm,
                 buf0, buf1, buf2, idx_v, prm_v, sub_v, win_v, sem,
                 sem_in0, sem_in1, sem_in2, sem_out0, sem_out1, sem_out2):
        wid = lax.axis_index("s") * _NC + lax.axis_index("c")
        pltpu.sync_copy(sub_hbm, sub_v)

        n_b = 2 * pieces_per_w
        bvals = []
        for k in range(pieces_per_w):
            p = wid + k * _NW
            bvals.append(p * _PIECE)
            bvals.append((p + 1) * _PIECE)

        def cnt_sub(i, carry):
            v = sub_v[pl.ds(i * _LANES, _LANES)]
            return tuple(
                carry[j] + plsc.all_reduce_population_count(v < bvals[j])[0]
                for j in range(n_b)
            )

        zero = jnp.int32(0)
        c1 = lax.fori_loop(0, n_sub // _LANES, cnt_sub, (zero,) * n_b)
        offs = [lax.max(c1[j] - 1, zero) * _SUB for j in range(n_b)]

        for j in range(n_b):
            pltpu.async_copy(
                idx_hbm.at[pl.ds(pl.multiple_of(offs[j], 8), _SUB)],
                win_v.at[j], sem)
        for j in range(n_b):
            pltpu.make_async_copy(
                idx_hbm.at[pl.ds(0, _SUB)], win_v.at[0], sem).wait()

        bounds = []
        for j in range(n_b):
            def cnt_win(i, carry, j=j):
                v = win_v[j, pl.ds(i * _LANES, _LANES)]
                return carry + plsc.all_reduce_population_count(
                    v < bvals[j])[0]
            bounds.append(
                offs[j] + lax.fori_loop(0, _SUB // _LANES, cnt_win, zero))

        sem_outs = [sem_out0, sem_out1, sem_out2]
        sem_ins = [sem_in0, sem_in1, sem_in2]
        ring = [buf0, buf1, buf2]
        _R = len(ring)

        def drain_out(slot):
            pltpu.make_async_copy(ring[slot], out_hbm.at[pl.ds(0, _PIECE)],
                                  sem_outs[slot]).wait()

        def piece_args(k):
            p = wid + k * _NW
            plo = p * _PIECE
            a = bounds[2 * k]
            b = bounds[2 * k + 1]
            cnt = b - a
            is_full = (cnt == _PIECE) & ((a & 7) == 0)
            return plo, a, b, cnt, is_full

        def fire_in(k):
            plo, a, b, cnt, is_full = piece_args(k)
            buf, s = ring[k % _R], sem_ins[k % _R]

            @pl.when(is_full)
            def _():
                pltpu.async_copy(
                    prm_hbm.at[pl.ds(pl.multiple_of(a, 8), _PIECE)], buf, s)

            @pl.when(jnp.logical_not(is_full))
            def _():
                pltpu.async_copy(fixed_hbm.at[pl.ds(plo, _PIECE)], buf, s)

        fire_in(0)
        for k in range(pieces_per_w):
            plo, a, b, cnt, is_full = piece_args(k)
            phi = plo + _PIECE
            buf = ring[k % _R]

            if k + 1 < pieces_per_w:
                if k + 1 >= _R:
                    drain_out((k + 1) % _R)
                fire_in(k + 1)

            pltpu.make_async_copy(fixed_hbm.at[pl.ds(0, _PIECE)], buf,
                                  sem_ins[k % _R]).wait()

            @pl.when((cnt > 0) & jnp.logical_not(is_full))
            def _():
                a_r = a & ~7
                n_chunks = (b - a_r + _CHUNK - 1) // _CHUNK

                def do_chunk(c, carry2):
                    base = pl.multiple_of(
                        lax.min(a_r + c * _CHUNK, r - _CHUNK), 8)
                    pltpu.sync_copy(idx_hbm.at[pl.ds(base, _CHUNK)], idx_v)
                    pltpu.sync_copy(prm_hbm.at[pl.ds(base, _CHUNK)], prm_v)
                    rem = b - base
                    n_vec = lax.min((rem + _LANES - 1) // _LANES,
                                    _CHUNK // _LANES)

                    def do_vec(v, carry3):
                        iv = idx_v[pl.ds(v * _LANES, _LANES)]
                        pv = prm_v[pl.ds(v * _LANES, _LANES)]
                        mask = (iv >= plo) & (iv < phi)
                        plsc.store_scatter(buf, [iv - plo], pv, mask=mask)
                        return carry3

                    lax.fori_loop(0, n_vec, do_vec, 0)
                    return carry2

                lax.fori_loop(0, n_chunks, do_chunk, 0)

            pltpu.async_copy(buf, out_hbm.at[pl.ds(plo, _PIECE)],
                             sem_outs[k % _R])

        for k in range(max(0, pieces_per_w - _R), pieces_per_w):
            drain_out(k % _R)

    return sc_merge(fixed_values, refinable_idx, refinable_params, sub)
